# baseline (speedup 1.0000x reference)
import functools

import jax
import jax.numpy as jnp
from jax.experimental import pallas as pl
from jax.experimental.pallas import tpu as pltpu

_VMEM_LIMIT = 48 * 1024 * 1024


def _conv_stats_kernel(x_ref, w_ref, y_ref, sum_ref, sqs_ref, slab_ref,
                       *, k, h, w):
    rows = h * w
    cin = x_ref.shape[1]
    xhwc = x_ref[0].reshape(cin, rows).astype(jnp.bfloat16).T
    slab_ref[...] = jnp.zeros_like(slab_ref)
    slab_ref[k // 2:k // 2 + h, k // 2:k // 2 + w, :] = xhwc.reshape(h, w, cin)
    slab = slab_ref[...]
    cols = [slab[di:di + h, dj:dj + w, :].reshape(rows, -1)
            for di in range(k) for dj in range(k)]
    xcol = jnp.concatenate(cols, axis=1)
    acc = jnp.dot(xcol, w_ref[...],
                  preferred_element_type=jnp.float32)
    ones8 = jnp.ones((8, rows), jnp.float32)
    sum_ref[0] = jnp.dot(ones8, acc, preferred_element_type=jnp.float32)
    sqs_ref[0] = jnp.dot(ones8, acc * acc, preferred_element_type=jnp.float32)
    y_ref[0] = acc.T.astype(jnp.bfloat16)


def _bn_mish_kernel(y_ref, sums_ref, sqs_ref, gamma_ref, beta_ref, o_ref,
                    *, count, eps):
    s = jnp.sum(sums_ref[:, 0, :], axis=0)
    q = jnp.sum(sqs_ref[:, 0, :], axis=0)
    mean = s / count
    var = jnp.maximum(q / count - mean * mean, 0.0)
    inv_std = jax.lax.rsqrt(var + eps)
    g = gamma_ref[0]
    scale = (g * inv_std).reshape(-1, 1)
    shift = (beta_ref[0] - mean * g * inv_std).reshape(-1, 1)
    z = y_ref[0].astype(jnp.float32) * scale + shift
    t = jnp.exp(jnp.minimum(z, 20.0))
    u = t * (t + 2.0)
    mish = z * u * pl.reciprocal(u + 2.0, approx=True)
    o_ref[0] = jnp.where(z > 20.0, z, mish).astype(o_ref.dtype)


def kernel(x_nchw, conv_w, gamma, beta):
    eps = 1e-5
    N, Cin, H, W = x_nchw.shape
    Cout, _, k, _ = conv_w.shape
    p = k // 2
    rows = H * W
    Hp, Wp = H + 2 * p, W + 2 * p

    w_flat = jnp.transpose(conv_w, (2, 3, 1, 0)).astype(jnp.bfloat16)
    w_flat = w_flat.reshape(k * k * Cin, Cout)

    conv_kernel = functools.partial(_conv_stats_kernel, k=k, h=H, w=W)
    y, sums, sqs = pl.pallas_call(
        conv_kernel,
        out_shape=(jax.ShapeDtypeStruct((N, Cout, rows), jnp.bfloat16),
                   jax.ShapeDtypeStruct((N, 8, Cout), jnp.float32),
                   jax.ShapeDtypeStruct((N, 8, Cout), jnp.float32)),
        grid=(N,),
        in_specs=[pl.BlockSpec((1, Cin, H, W), lambda n: (n, 0, 0, 0)),
                  pl.BlockSpec((k * k * Cin, Cout), lambda n: (0, 0))],
        out_specs=(pl.BlockSpec((1, Cout, rows), lambda n: (n, 0, 0)),
                   pl.BlockSpec((1, 8, Cout), lambda n: (n, 0, 0)),
                   pl.BlockSpec((1, 8, Cout), lambda n: (n, 0, 0))),
        scratch_shapes=[pltpu.VMEM((Hp, Wp, Cin), jnp.bfloat16)],
        compiler_params=pltpu.CompilerParams(
            dimension_semantics=("parallel",),
            vmem_limit_bytes=_VMEM_LIMIT),
    )(x_nchw, w_flat)

    bn_kernel = functools.partial(_bn_mish_kernel, count=float(N * rows),
                                  eps=eps)
    out_flat = pl.pallas_call(
        bn_kernel,
        out_shape=jax.ShapeDtypeStruct((N, Cout, rows), jnp.float32),
        grid=(N,),
        in_specs=[pl.BlockSpec((1, Cout, rows), lambda n: (n, 0, 0)),
                  pl.BlockSpec((N, 8, Cout), lambda n: (0, 0, 0)),
                  pl.BlockSpec((N, 8, Cout), lambda n: (0, 0, 0)),
                  pl.BlockSpec((1, Cout), lambda n: (0, 0)),
                  pl.BlockSpec((1, Cout), lambda n: (0, 0))],
        out_specs=pl.BlockSpec((1, Cout, rows), lambda n: (n, 0, 0)),
        compiler_params=pltpu.CompilerParams(
            dimension_semantics=("parallel",),
            vmem_limit_bytes=_VMEM_LIMIT),
    )(y, sums, sqs, gamma.reshape(1, Cout).astype(jnp.float32),
      beta.reshape(1, Cout).astype(jnp.float32))

    return out_flat.reshape(N, Cout, H, W)


# speedup vs baseline: 1.1115x; 1.1115x over previous
import functools

import jax
import jax.numpy as jnp
from jax.experimental import pallas as pl
from jax.experimental.pallas import tpu as pltpu

_VMEM_LIMIT = 48 * 1024 * 1024


def _conv_stats_kernel(x_ref, w_ref, y_ref, sum_ref, sqs_ref, slab_ref,
                       *, k, h, w):
    rows = h * w
    cin = x_ref.shape[1]
    xhwc = x_ref[0].astype(jnp.bfloat16).T
    slab_ref[...] = jnp.zeros_like(slab_ref)
    slab_ref[k // 2:k // 2 + h, k // 2:k // 2 + w, :] = xhwc.reshape(h, w, cin)
    slab = slab_ref[...]
    cols = [slab[di:di + h, dj:dj + w, :].reshape(rows, -1)
            for di in range(k) for dj in range(k)]
    xcol = jnp.concatenate(cols, axis=1)
    acc = jnp.dot(xcol, w_ref[...],
                  preferred_element_type=jnp.float32)
    ones8 = jnp.ones((8, rows), jnp.float32)
    sum_ref[0] = jnp.dot(ones8, acc, preferred_element_type=jnp.float32)
    sqs_ref[0] = jnp.dot(ones8, acc * acc, preferred_element_type=jnp.float32)
    y_ref[0] = acc.T.astype(jnp.bfloat16)


def _bn_mish_kernel(y_ref, sums_ref, sqs_ref, gamma_ref, beta_ref, o_ref,
                    *, count, eps):
    s = jnp.sum(sums_ref[:, 0, :], axis=0)
    q = jnp.sum(sqs_ref[:, 0, :], axis=0)
    mean = s / count
    var = jnp.maximum(q / count - mean * mean, 0.0)
    inv_std = jax.lax.rsqrt(var + eps)
    g = gamma_ref[0]
    scale = (g * inv_std).reshape(-1, 1)
    shift = (beta_ref[0] - mean * g * inv_std).reshape(-1, 1)
    z = y_ref[0].astype(jnp.float32) * scale + shift
    t = jnp.exp(jnp.minimum(z, 20.0))
    u = t * (t + 2.0)
    mish = z * u * pl.reciprocal(u + 2.0, approx=True)
    res = jnp.where(z > 20.0, z, mish)
    o_ref[0] = res.reshape(o_ref.shape[1:]).astype(o_ref.dtype)


def kernel(x_nchw, conv_w, gamma, beta):
    eps = 1e-5
    N, Cin, H, W = x_nchw.shape
    Cout, _, k, _ = conv_w.shape
    p = k // 2
    rows = H * W
    Hp, Wp = H + 2 * p, W + 2 * p

    w_flat = jnp.transpose(conv_w, (2, 3, 1, 0)).astype(jnp.bfloat16)
    w_flat = w_flat.reshape(k * k * Cin, Cout)

    x_flat = x_nchw.reshape(N, Cin, rows)

    conv_kernel = functools.partial(_conv_stats_kernel, k=k, h=H, w=W)
    y, sums, sqs = pl.pallas_call(
        conv_kernel,
        out_shape=(jax.ShapeDtypeStruct((N, Cout, rows), jnp.bfloat16),
                   jax.ShapeDtypeStruct((N, 8, Cout), jnp.float32),
                   jax.ShapeDtypeStruct((N, 8, Cout), jnp.float32)),
        grid=(N,),
        in_specs=[pl.BlockSpec((1, Cin, rows), lambda n: (n, 0, 0)),
                  pl.BlockSpec((k * k * Cin, Cout), lambda n: (0, 0))],
        out_specs=(pl.BlockSpec((1, Cout, rows), lambda n: (n, 0, 0)),
                   pl.BlockSpec((1, 8, Cout), lambda n: (n, 0, 0)),
                   pl.BlockSpec((1, 8, Cout), lambda n: (n, 0, 0))),
        scratch_shapes=[pltpu.VMEM((Hp, Wp, Cin), jnp.bfloat16)],
        compiler_params=pltpu.CompilerParams(
            dimension_semantics=("parallel",),
            vmem_limit_bytes=_VMEM_LIMIT),
    )(x_flat, w_flat)

    bn_kernel = functools.partial(_bn_mish_kernel, count=float(N * rows),
                                  eps=eps)
    out = pl.pallas_call(
        bn_kernel,
        out_shape=jax.ShapeDtypeStruct((N, Cout, H, W), jnp.float32),
        grid=(N,),
        in_specs=[pl.BlockSpec((1, Cout, rows), lambda n: (n, 0, 0)),
                  pl.BlockSpec((N, 8, Cout), lambda n: (0, 0, 0)),
                  pl.BlockSpec((N, 8, Cout), lambda n: (0, 0, 0)),
                  pl.BlockSpec((1, Cout), lambda n: (0, 0)),
                  pl.BlockSpec((1, Cout), lambda n: (0, 0))],
        out_specs=pl.BlockSpec((1, Cout, H, W), lambda n: (n, 0, 0, 0)),
        compiler_params=pltpu.CompilerParams(
            dimension_semantics=("parallel",),
            vmem_limit_bytes=_VMEM_LIMIT),
    )(y, sums, sqs, gamma.reshape(1, Cout).astype(jnp.float32),
      beta.reshape(1, Cout).astype(jnp.float32))

    return out


# speedup vs baseline: 1.4126x; 1.2708x over previous
import functools

import jax
import jax.numpy as jnp
from jax.experimental import pallas as pl
from jax.experimental.pallas import tpu as pltpu

_VMEM_LIMIT = 48 * 1024 * 1024


def _conv_stats_kernel(x_ref, w_ref, y_ref, sum_ref, sqs_ref, *, k, h, w):
    rows = h * w
    slab = x_ref[0]
    cols = [slab[di:di + h, dj:dj + w, :].reshape(rows, -1)
            for di in range(k) for dj in range(k)]
    xcol = jnp.concatenate(cols, axis=1)
    acc = jnp.dot(xcol, w_ref[...],
                  preferred_element_type=jnp.float32)
    ones8 = jnp.ones((8, rows), jnp.float32)
    sum_ref[0] = jnp.dot(ones8, acc, preferred_element_type=jnp.float32)
    sqs_ref[0] = jnp.dot(ones8, acc * acc, preferred_element_type=jnp.float32)
    y_ref[0] = acc.T.astype(jnp.bfloat16)


def _bn_mish_kernel(y_ref, sums_ref, sqs_ref, gamma_ref, beta_ref, o_ref,
                    *, count, eps):
    s = jnp.sum(sums_ref[:, 0, :], axis=0)
    q = jnp.sum(sqs_ref[:, 0, :], axis=0)
    mean = s / count
    var = jnp.maximum(q / count - mean * mean, 0.0)
    inv_std = jax.lax.rsqrt(var + eps)
    g = gamma_ref[0]
    scale = (g * inv_std).reshape(-1, 1)
    shift = (beta_ref[0] - mean * g * inv_std).reshape(-1, 1)
    z = y_ref[0].astype(jnp.float32) * scale + shift
    t = jnp.exp(jnp.minimum(z, 20.0))
    u = t * (t + 2.0)
    mish = z * u * pl.reciprocal(u + 2.0, approx=True)
    o_ref[0] = jnp.where(z > 20.0, z, mish).astype(o_ref.dtype)


def kernel(x_nchw, conv_w, gamma, beta):
    eps = 1e-5
    N, Cin, H, W = x_nchw.shape
    Cout, _, k, _ = conv_w.shape
    p = k // 2
    rows = H * W
    Hp, Wp = H + 2 * p, W + 2 * p

    x_nhwc = jnp.transpose(x_nchw, (0, 2, 3, 1)).astype(jnp.bfloat16)
    xp = jnp.pad(x_nhwc, ((0, 0), (p, p), (p, p), (0, 0)))

    w_flat = jnp.transpose(conv_w, (2, 3, 1, 0)).astype(jnp.bfloat16)
    w_flat = w_flat.reshape(k * k * Cin, Cout)

    conv_kernel = functools.partial(_conv_stats_kernel, k=k, h=H, w=W)
    y, sums, sqs = pl.pallas_call(
        conv_kernel,
        out_shape=(jax.ShapeDtypeStruct((N, Cout, rows), jnp.bfloat16),
                   jax.ShapeDtypeStruct((N, 8, Cout), jnp.float32),
                   jax.ShapeDtypeStruct((N, 8, Cout), jnp.float32)),
        grid=(N,),
        in_specs=[pl.BlockSpec((1, Hp, Wp, Cin), lambda n: (n, 0, 0, 0)),
                  pl.BlockSpec((k * k * Cin, Cout), lambda n: (0, 0))],
        out_specs=(pl.BlockSpec((1, Cout, rows), lambda n: (n, 0, 0)),
                   pl.BlockSpec((1, 8, Cout), lambda n: (n, 0, 0)),
                   pl.BlockSpec((1, 8, Cout), lambda n: (n, 0, 0))),
        compiler_params=pltpu.CompilerParams(
            dimension_semantics=("parallel",),
            vmem_limit_bytes=_VMEM_LIMIT),
    )(xp, w_flat)

    bn_kernel = functools.partial(_bn_mish_kernel, count=float(N * rows),
                                  eps=eps)
    out_flat = pl.pallas_call(
        bn_kernel,
        out_shape=jax.ShapeDtypeStruct((N, Cout, rows), jnp.float32),
        grid=(N,),
        in_specs=[pl.BlockSpec((1, Cout, rows), lambda n: (n, 0, 0)),
                  pl.BlockSpec((N, 8, Cout), lambda n: (0, 0, 0)),
                  pl.BlockSpec((N, 8, Cout), lambda n: (0, 0, 0)),
                  pl.BlockSpec((1, Cout), lambda n: (0, 0)),
                  pl.BlockSpec((1, Cout), lambda n: (0, 0))],
        out_specs=pl.BlockSpec((1, Cout, rows), lambda n: (n, 0, 0)),
        compiler_params=pltpu.CompilerParams(
            dimension_semantics=("parallel",),
            vmem_limit_bytes=_VMEM_LIMIT),
    )(y, sums, sqs, gamma.reshape(1, Cout).astype(jnp.float32),
      beta.reshape(1, Cout).astype(jnp.float32))

    return out_flat.reshape(N, Cout, H, W)


# speedup vs baseline: 1.4475x; 1.0248x over previous
import functools

import jax
import jax.numpy as jnp
from jax.experimental import pallas as pl
from jax.experimental.pallas import tpu as pltpu

_VMEM_LIMIT = 48 * 1024 * 1024


def _conv_stats_kernel(x_ref, w_ref, y_ref, sum_ref, sqs_ref, *, k, h, w):
    rows = h * w
    slab = x_ref[0]
    cols = [slab[di:di + h, dj:dj + w, :].reshape(rows, -1)
            for di in range(k) for dj in range(k)]
    xcol = jnp.concatenate(cols, axis=1)
    acc = jnp.dot(xcol, w_ref[...],
                  preferred_element_type=jnp.float32)
    ones8 = jnp.ones((8, rows), jnp.float32)
    sum_ref[0] = jnp.dot(ones8, acc, preferred_element_type=jnp.float32)
    sqs_ref[0] = jnp.dot(ones8, acc * acc, preferred_element_type=jnp.float32)
    y_ref[0] = acc.T.astype(jnp.bfloat16)


def _bn_mish_kernel(y_ref, sums_ref, sqs_ref, gamma_ref, beta_ref, o_ref,
                    *, count, eps):
    s = jnp.sum(sums_ref[:, 0, :], axis=0)
    q = jnp.sum(sqs_ref[:, 0, :], axis=0)
    mean = s / count
    var = jnp.maximum(q / count - mean * mean, 0.0)
    inv_std = jax.lax.rsqrt(var + eps)
    g = gamma_ref[0]
    scale = (g * inv_std).reshape(-1, 1)
    shift = (beta_ref[0] - mean * g * inv_std).reshape(-1, 1)
    z = y_ref[0].astype(jnp.float32) * scale + shift
    t = jnp.exp(jnp.minimum(z, 20.0))
    u = t * (t + 2.0)
    mish = z * u * pl.reciprocal(u + 2.0, approx=True)
    o_ref[0] = jnp.where(z > 20.0, z, mish).astype(o_ref.dtype)


def kernel(x_nchw, conv_w, gamma, beta):
    eps = 1e-5
    N, Cin, H, W = x_nchw.shape
    Cout, _, k, _ = conv_w.shape
    p = k // 2
    rows = H * W
    Hp, Wp = H + 2 * p, W + 2 * p

    x_nhwc = jnp.transpose(x_nchw, (0, 2, 3, 1)).astype(jnp.bfloat16)
    xp = jnp.pad(x_nhwc, ((0, 0), (p, p), (p, p), (0, 0)))

    w_flat = jnp.transpose(conv_w, (2, 3, 1, 0)).astype(jnp.bfloat16)
    w_flat = w_flat.reshape(k * k * Cin, Cout)

    conv_kernel = functools.partial(_conv_stats_kernel, k=k, h=H, w=W)
    y, sums, sqs = pl.pallas_call(
        conv_kernel,
        out_shape=(jax.ShapeDtypeStruct((N, Cout, rows), jnp.bfloat16),
                   jax.ShapeDtypeStruct((N, 8, Cout), jnp.float32),
                   jax.ShapeDtypeStruct((N, 8, Cout), jnp.float32)),
        grid=(N,),
        in_specs=[pl.BlockSpec((1, Hp, Wp, Cin), lambda n: (n, 0, 0, 0)),
                  pl.BlockSpec((k * k * Cin, Cout), lambda n: (0, 0))],
        out_specs=(pl.BlockSpec((1, Cout, rows), lambda n: (n, 0, 0)),
                   pl.BlockSpec((1, 8, Cout), lambda n: (n, 0, 0)),
                   pl.BlockSpec((1, 8, Cout), lambda n: (n, 0, 0))),
        compiler_params=pltpu.CompilerParams(
            dimension_semantics=("parallel",),
            allow_input_fusion=[True, False],
            vmem_limit_bytes=_VMEM_LIMIT),
    )(xp, w_flat)

    bn_kernel = functools.partial(_bn_mish_kernel, count=float(N * rows),
                                  eps=eps)
    out_flat = pl.pallas_call(
        bn_kernel,
        out_shape=jax.ShapeDtypeStruct((N, Cout, rows), jnp.float32),
        grid=(N,),
        in_specs=[pl.BlockSpec((1, Cout, rows), lambda n: (n, 0, 0)),
                  pl.BlockSpec((N, 8, Cout), lambda n: (0, 0, 0)),
                  pl.BlockSpec((N, 8, Cout), lambda n: (0, 0, 0)),
                  pl.BlockSpec((1, Cout), lambda n: (0, 0)),
                  pl.BlockSpec((1, Cout), lambda n: (0, 0))],
        out_specs=pl.BlockSpec((1, Cout, rows), lambda n: (n, 0, 0)),
        compiler_params=pltpu.CompilerParams(
            dimension_semantics=("parallel",),
            vmem_limit_bytes=_VMEM_LIMIT),
    )(y, sums, sqs, gamma.reshape(1, Cout).astype(jnp.float32),
      beta.reshape(1, Cout).astype(jnp.float32))

    return out_flat.reshape(N, Cout, H, W)
